# baseline (speedup 1.0000x reference)
import functools

import jax
import jax.numpy as jnp
from jax import lax
from jax.experimental import pallas as pl
from jax.experimental.pallas import tpu as pltpu
from jax.experimental.pallas import tpu_sc as plsc

B = 32
SH, SW = 32, 32
L = 128
D = 1024
T = SH * SW + L
CHUNKS = (4, 4, 4, 4, 4, 4, 4, 4)
NPG = 16
NBG = 2
PPW = T // NPG
CR = 24
NCI = PPW // CR
NR = 4


def _sc_gather(xidx, embedding, bk):
    mesh = plsc.VectorSubcoreMesh(core_axis_name="c", subcore_axis_name="s")
    bpw = bk // NBG
    nt = bpw * NCI

    @functools.partial(
        pl.kernel,
        mesh=mesh,
        out_type=jax.ShapeDtypeStruct((bk, T, D), jnp.float32),
        scratch_types=[
            pltpu.VMEM((nt, CR), jnp.int32),
            pltpu.VMEM((CR, D), jnp.float32),
            pltpu.VMEM((CR, D), jnp.float32),
            pltpu.VMEM((CR, D), jnp.float32),
            pltpu.VMEM((CR, D), jnp.float32),
            pltpu.SemaphoreType.DMA,
            pltpu.SemaphoreType.DMA,
            pltpu.SemaphoreType.DMA,
            pltpu.SemaphoreType.DMA,
            pltpu.SemaphoreType.DMA,
            pltpu.SemaphoreType.DMA,
            pltpu.SemaphoreType.DMA,
            pltpu.SemaphoreType.DMA,
        ],
    )
    def k(x_hbm, emb_hbm, out_hbm,
          idx_v, buf0, buf1, buf2, buf3,
          sg0, sg1, sg2, sg3, sw0, sw1, sw2, sw3):
        pg = lax.axis_index("s")
        bg = lax.axis_index("c")
        p0 = pl.multiple_of(pg * PPW, 8)

        bufs = (buf0, buf1, buf2, buf3)
        sgat = (sg0, sg1, sg2, sg3)
        swri = (sw0, sw1, sw2, sw3)

        pltpu.sync_copy(x_hbm.at[pg, bg], idx_v)

        def issue_gather(t, r):
            pltpu.async_copy(emb_hbm.at[idx_v.at[t]], bufs[r], sgat[r])

        def wait_gather(r):
            pltpu.make_async_copy(emb_hbm.at[idx_v.at[0]],
                                  bufs[r], sgat[r]).wait()

        def issue_write(t, r):
            bl = t // NCI
            ci = t % NCI
            bglob = bg * bpw + bl
            off = pl.multiple_of(p0 + ci * CR, 8)
            pltpu.async_copy(bufs[r], out_hbm.at[bglob, pl.ds(off, CR)],
                             swri[r])

        def wait_write(r):
            pltpu.make_async_copy(bufs[r], out_hbm.at[0, pl.ds(0, CR)],
                                  swri[r]).wait()

        def slot(t, r):
            @pl.when(t + (NR - 1) < nt)
            def _():
                @pl.when(t >= 1)
                def _():
                    wait_write((r + NR - 1) % NR)
                issue_gather(t + (NR - 1), (r + NR - 1) % NR)

            wait_gather(r)
            issue_write(t, r)

        for t in range(min(NR - 1, nt)):
            issue_gather(t, t)

        def ibody(i, c):
            for r in range(NR):
                slot(NR * i + r, r)
            return c

        lax.fori_loop(0, nt // NR, ibody, 0)
        for j in range(nt % NR):
            t = (nt // NR) * NR + j
            slot(jnp.int32(t), t % NR)
        for r in range(min(NR, nt)):
            wait_write(r)

    return k(xidx, embedding)


def _pos_body(h_ref, w_ref, l_ref, out_ref):
    hh = jnp.broadcast_to(h_ref[...][:, None, :], (SH, SW, D))
    ww = jnp.broadcast_to(w_ref[...][None, :, :], (SH, SW, D))
    out_ref[: SH * SW, :] = (hh + ww).reshape(SH * SW, D)
    out_ref[SH * SW :, :] = l_ref[...]


def _tc_pos(height_emb, width_emb, length_emb):
    return pl.pallas_call(
        _pos_body,
        out_shape=jax.ShapeDtypeStruct((T, D), jnp.float32),
    )(height_emb, width_emb, length_emb)


def _add_first_body(g_ref, p_ref, out_ref):
    out_ref[...] = g_ref[...] + p_ref[...][None]


def _add_next_body(prev_ref, g_ref, p_ref, out_ref):
    del prev_ref
    out_ref[...] = g_ref[...] + p_ref[...][None]


def _tc_add(out_prev, g, pos, bk, ob):
    if out_prev is None:
        return pl.pallas_call(
            _add_first_body,
            grid=(bk,),
            in_specs=[
                pl.BlockSpec((1, T, D), lambda b: (b, 0, 0)),
                pl.BlockSpec((T, D), lambda b: (0, 0)),
            ],
            out_specs=pl.BlockSpec((1, T, D), lambda b: (b, 0, 0)),
            out_shape=jax.ShapeDtypeStruct((B, T, D), jnp.float32),
        )(g, pos)
    return pl.pallas_call(
        _add_next_body,
        grid=(bk,),
        in_specs=[
            pl.BlockSpec(memory_space=pl.ANY),
            pl.BlockSpec((1, T, D), lambda b: (b, 0, 0)),
            pl.BlockSpec((T, D), lambda b: (0, 0)),
        ],
        out_specs=pl.BlockSpec((1, T, D),
                               lambda b, ob=ob: (ob + b, 0, 0)),
        out_shape=jax.ShapeDtypeStruct((B, T, D), jnp.float32),
        input_output_aliases={0: 0},
    )(out_prev, g, pos)


@jax.jit
def kernel(x, embedding, height_emb, width_emb, length_emb):
    xi = x.astype(jnp.int32)
    pos = _tc_pos(height_emb, width_emb, length_emb)
    gs = []
    ob = 0
    for bk in CHUNKS:
        bpw = bk // NBG
        xk = (xi[ob:ob + bk]
              .reshape(NBG, bpw, NPG, NCI, CR).transpose(2, 0, 1, 3, 4)
              .reshape(NPG, NBG, bpw * NCI, CR))
        gs.append(_sc_gather(xk, embedding, bk))
        ob += bk
    out = None
    ob = 0
    for bk, g in zip(CHUNKS, gs):
        out = _tc_add(out, g, pos, bk, ob)
        ob += bk
    return out


# speedup vs baseline: 1.0636x; 1.0636x over previous
import functools

import jax
import jax.numpy as jnp
from jax import lax
from jax.experimental import pallas as pl
from jax.experimental.pallas import tpu as pltpu
from jax.experimental.pallas import tpu_sc as plsc

B = 32
SH, SW = 32, 32
L = 128
D = 1024
T = SH * SW + L
CHUNKS = (16, 16)
NPG = 16
NBG = 2
PPW = T // NPG
CR = 24
NCI = PPW // CR
NR = 4


def _sc_gather(xidx, embedding, bk):
    mesh = plsc.VectorSubcoreMesh(core_axis_name="c", subcore_axis_name="s")
    bpw = bk // NBG
    nt = bpw * NCI

    @functools.partial(
        pl.kernel,
        mesh=mesh,
        out_type=jax.ShapeDtypeStruct((bk, T, D), jnp.float32),
        scratch_types=[
            pltpu.VMEM((nt, CR), jnp.int32),
            pltpu.VMEM((CR, D), jnp.float32),
            pltpu.VMEM((CR, D), jnp.float32),
            pltpu.VMEM((CR, D), jnp.float32),
            pltpu.VMEM((CR, D), jnp.float32),
            pltpu.SemaphoreType.DMA,
            pltpu.SemaphoreType.DMA,
            pltpu.SemaphoreType.DMA,
            pltpu.SemaphoreType.DMA,
            pltpu.SemaphoreType.DMA,
            pltpu.SemaphoreType.DMA,
            pltpu.SemaphoreType.DMA,
            pltpu.SemaphoreType.DMA,
        ],
    )
    def k(x_hbm, emb_hbm, out_hbm,
          idx_v, buf0, buf1, buf2, buf3,
          sg0, sg1, sg2, sg3, sw0, sw1, sw2, sw3):
        pg = lax.axis_index("s")
        bg = lax.axis_index("c")
        p0 = pl.multiple_of(pg * PPW, 8)

        bufs = (buf0, buf1, buf2, buf3)
        sgat = (sg0, sg1, sg2, sg3)
        swri = (sw0, sw1, sw2, sw3)

        pltpu.sync_copy(x_hbm.at[pg, bg], idx_v)

        def issue_gather(t, r):
            pltpu.async_copy(emb_hbm.at[idx_v.at[t]], bufs[r], sgat[r])

        def wait_gather(r):
            pltpu.make_async_copy(emb_hbm.at[idx_v.at[0]],
                                  bufs[r], sgat[r]).wait()

        def issue_write(t, r):
            bl = t // NCI
            ci = t % NCI
            bglob = bg * bpw + bl
            off = pl.multiple_of(p0 + ci * CR, 8)
            pltpu.async_copy(bufs[r], out_hbm.at[bglob, pl.ds(off, CR)],
                             swri[r])

        def wait_write(r):
            pltpu.make_async_copy(bufs[r], out_hbm.at[0, pl.ds(0, CR)],
                                  swri[r]).wait()

        def slot(t, r):
            @pl.when(t + (NR - 1) < nt)
            def _():
                @pl.when(t >= 1)
                def _():
                    wait_write((r + NR - 1) % NR)
                issue_gather(t + (NR - 1), (r + NR - 1) % NR)

            wait_gather(r)
            issue_write(t, r)

        for t in range(min(NR - 1, nt)):
            issue_gather(t, t)

        def ibody(i, c):
            for r in range(NR):
                slot(NR * i + r, r)
            return c

        lax.fori_loop(0, nt // NR, ibody, 0)
        for j in range(nt % NR):
            t = (nt // NR) * NR + j
            slot(jnp.int32(t), t % NR)
        for r in range(min(NR, nt)):
            wait_write(r)

    return k(xidx, embedding)


def _pos_body(h_ref, w_ref, l_ref, out_ref):
    hh = jnp.broadcast_to(h_ref[...][:, None, :], (SH, SW, D))
    ww = jnp.broadcast_to(w_ref[...][None, :, :], (SH, SW, D))
    out_ref[: SH * SW, :] = (hh + ww).reshape(SH * SW, D)
    out_ref[SH * SW :, :] = l_ref[...]


def _tc_pos(height_emb, width_emb, length_emb):
    return pl.pallas_call(
        _pos_body,
        out_shape=jax.ShapeDtypeStruct((T, D), jnp.float32),
    )(height_emb, width_emb, length_emb)


def _add_first_body(g_ref, p_ref, out_ref):
    out_ref[...] = g_ref[...] + p_ref[...][None]


def _add_next_body(prev_ref, g_ref, p_ref, out_ref):
    del prev_ref
    out_ref[...] = g_ref[...] + p_ref[...][None]


def _tc_add(out_prev, g, pos, bk, ob):
    if out_prev is None:
        return pl.pallas_call(
            _add_first_body,
            grid=(bk,),
            in_specs=[
                pl.BlockSpec((1, T, D), lambda b: (b, 0, 0)),
                pl.BlockSpec((T, D), lambda b: (0, 0)),
            ],
            out_specs=pl.BlockSpec((1, T, D), lambda b: (b, 0, 0)),
            out_shape=jax.ShapeDtypeStruct((B, T, D), jnp.float32),
        )(g, pos)
    return pl.pallas_call(
        _add_next_body,
        grid=(bk,),
        in_specs=[
            pl.BlockSpec(memory_space=pl.ANY),
            pl.BlockSpec((1, T, D), lambda b: (b, 0, 0)),
            pl.BlockSpec((T, D), lambda b: (0, 0)),
        ],
        out_specs=pl.BlockSpec((1, T, D),
                               lambda b, ob=ob: (ob + b, 0, 0)),
        out_shape=jax.ShapeDtypeStruct((B, T, D), jnp.float32),
        input_output_aliases={0: 0},
    )(out_prev, g, pos)


@jax.jit
def kernel(x, embedding, height_emb, width_emb, length_emb):
    xi = x.astype(jnp.int32)
    pos = _tc_pos(height_emb, width_emb, length_emb)
    gs = []
    ob = 0
    for bk in CHUNKS:
        bpw = bk // NBG
        xk = (xi[ob:ob + bk]
              .reshape(NBG, bpw, NPG, NCI, CR).transpose(2, 0, 1, 3, 4)
              .reshape(NPG, NBG, bpw * NCI, CR))
        gs.append(_sc_gather(xk, embedding, bk))
        ob += bk
    out = None
    ob = 0
    for bk, g in zip(CHUNKS, gs):
        out = _tc_add(out, g, pos, bk, ob)
        ob += bk
    return out
